# baseline (speedup 1.0000x reference)
import functools

import jax
import jax.numpy as jnp
from jax import lax
from jax.experimental import pallas as pl
from jax.experimental.pallas import tpu as pltpu
from jax.experimental.pallas import tpu_sc as plsc

_V = 1024
_D = 3
_N = 16384

_NC = 2
_NS = 16
_L = 16
_NW = _NC * _NS
_BPW = _N // _NW
_RPS = _V // _NS
_OB = 256
_NOB = _BPW // _OB


def _gather_call(table, idx):
  mesh = plsc.VectorSubcoreMesh(core_axis_name="c", subcore_axis_name="s")

  @functools.partial(
      pl.kernel,
      mesh=mesh,
      out_type=jax.ShapeDtypeStruct((_N, _D), jnp.float32),
      compiler_params=pltpu.CompilerParams(
          needs_layout_passes=False, skip_device_barrier=True),
      scratch_types=[
          pltpu.VMEM((_RPS, _D), jnp.float32),
          pltpu.VMEM((_V * _D,), jnp.float32),
          pltpu.VMEM((_BPW,), jnp.int32),
          pltpu.VMEM((2, _OB, _D), jnp.float32),
          pltpu.VMEM_SHARED((_V * _D,), jnp.float32),
          pltpu.SemaphoreType.DMA,
          pltpu.SemaphoreType.DMA((2,)),
      ],
  )
  def k(table_hbm, idx_hbm, out_hbm, stage_v, tab_v, idx_v, outb_v,
        tab_s, sem_i, sem_o):
    cid = lax.axis_index("c")
    sid = lax.axis_index("s")
    wid = sid * _NC + cid
    base = wid * _BPW
    cp_i = pltpu.async_copy(idx_hbm.at[pl.ds(base, _BPW)], idx_v, sem_i)
    loc = lax.iota(jnp.int32, _L)

    row0 = sid * _RPS
    pltpu.sync_copy(table_hbm.at[pl.ds(row0, _RPS)], stage_v)
    dst0 = row0 * _D
    for g in range(_RPS // _L):
      r16 = loc + g * _L
      pos = dst0 + (loc + g * _L) * _D
      for c in range(_D):
        cc = jnp.full((_L,), c, jnp.int32)
        v = plsc.load_gather(stage_v, [r16, cc])
        plsc.store_scatter(tab_v, [pos + c], v)
    pltpu.sync_copy(
        tab_v.at[pl.ds(dst0, _RPS * _D)], tab_s.at[pl.ds(dst0, _RPS * _D)])
    plsc.subcore_barrier()
    pltpu.sync_copy(tab_s, tab_v)

    cp_i.wait()
    cp_o = [None, None]
    for ob in range(_NOB):
      buf = ob % 2
      if cp_o[buf] is not None:
        cp_o[buf].wait()
      outb = outb_v.at[buf]
      for g in range(_OB // _L):
        rows = idx_v[pl.ds(ob * _OB + g * _L, _L)]
        offs = rows * _D
        pos = loc + g * _L
        for c in range(_D):
          cc = jnp.full((_L,), c, jnp.int32)
          col = plsc.load_gather(tab_v, [offs + c])
          plsc.store_scatter(outb, [pos, cc], col)
      cp_o[buf] = pltpu.async_copy(
          outb, out_hbm.at[pl.ds(base + ob * _OB, _OB)], sem_o.at[buf])
    cp_o[0].wait()
    cp_o[1].wait()

  return k(table, idx)


def kernel(embedding, indices):
  return _gather_call(embedding, indices)


# speedup vs baseline: 1.0090x; 1.0090x over previous
import functools

import jax
import jax.numpy as jnp
from jax import lax
from jax.experimental import pallas as pl
from jax.experimental.pallas import tpu as pltpu
from jax.experimental.pallas import tpu_sc as plsc

_V = 1024
_D = 3
_N = 16384

_NC = 2
_NS = 16
_L = 16
_NW = _NC * _NS
_BPW = _N // _NW
_RPS = _V // _NS
_OB = 128
_NOB = _BPW // _OB


def _gather_call(table, idx):
  mesh = plsc.VectorSubcoreMesh(core_axis_name="c", subcore_axis_name="s")

  @functools.partial(
      pl.kernel,
      mesh=mesh,
      out_type=jax.ShapeDtypeStruct((_N, _D), jnp.float32),
      compiler_params=pltpu.CompilerParams(
          needs_layout_passes=False, skip_device_barrier=True),
      scratch_types=[
          pltpu.VMEM((_RPS, _D), jnp.float32),
          pltpu.VMEM((_V * _D,), jnp.float32),
          pltpu.VMEM((_BPW,), jnp.int32),
          pltpu.VMEM((2, _OB, _D), jnp.float32),
          pltpu.VMEM_SHARED((_V * _D,), jnp.float32),
          pltpu.SemaphoreType.DMA,
          pltpu.SemaphoreType.DMA((2,)),
      ],
  )
  def k(table_hbm, idx_hbm, out_hbm, stage_v, tab_v, idx_v, outb_v,
        tab_s, sem_i, sem_o):
    cid = lax.axis_index("c")
    sid = lax.axis_index("s")
    wid = sid * _NC + cid
    base = wid * _BPW
    cp_i = pltpu.async_copy(idx_hbm.at[pl.ds(base, _BPW)], idx_v, sem_i)
    loc = lax.iota(jnp.int32, _L)

    row0 = sid * _RPS
    pltpu.sync_copy(table_hbm.at[pl.ds(row0, _RPS)], stage_v)
    dst0 = row0 * _D
    for g in range(_RPS // _L):
      r16 = loc + g * _L
      pos = dst0 + (loc + g * _L) * _D
      for c in range(_D):
        cc = jnp.full((_L,), c, jnp.int32)
        v = plsc.load_gather(stage_v, [r16, cc])
        plsc.store_scatter(tab_v, [pos + c], v)
    pltpu.sync_copy(
        tab_v.at[pl.ds(dst0, _RPS * _D)], tab_s.at[pl.ds(dst0, _RPS * _D)])
    plsc.subcore_barrier()
    pltpu.sync_copy(tab_s, tab_v)

    cp_i.wait()
    cp_o = [None, None]
    for ob in range(_NOB):
      buf = ob % 2
      if cp_o[buf] is not None:
        cp_o[buf].wait()
      outb = outb_v.at[buf]
      for g in range(_OB // _L):
        rows = idx_v[pl.ds(ob * _OB + g * _L, _L)]
        offs = rows * _D
        pos = loc + g * _L
        for c in range(_D):
          cc = jnp.full((_L,), c, jnp.int32)
          col = plsc.load_gather(tab_v, [offs + c])
          plsc.store_scatter(outb, [pos, cc], col)
      cp_o[buf] = pltpu.async_copy(
          outb, out_hbm.at[pl.ds(base + ob * _OB, _OB)], sem_o.at[buf])
    cp_o[0].wait()
    cp_o[1].wait()

  return k(table, idx)


def kernel(embedding, indices):
  return _gather_call(embedding, indices)
